# Initial kernel scaffold; baseline (speedup 1.0000x reference)
import functools

import jax
import jax.numpy as jnp
from jax import lax
from jax.experimental import pallas as pl
from jax.experimental.pallas import tpu as pltpu
from jax.experimental.pallas import tpu_sc as plsc

N = 10000
D = 128
NC = 2
NS = 16
NW = NC * NS
CH = 128
E = 320000
CHUNKS = -(-E // (NW * CH))
E_PAD = NW * CHUNKS * CH
NP = N + 16
RPS = NP // NS
NPD = 10240
RPD = NPD // NS

_mesh = lambda: plsc.VectorSubcoreMesh(core_axis_name="c", subcore_axis_name="s")


def _deg_body(dst_hbm, z_hbm, out_hbm, idx_v, ones_v, acc):
    c = lax.axis_index("c")
    s = lax.axis_index("s")
    w = c * NS + s
    pltpu.sync_copy(z_hbm.at[pl.ds(s * RPD, RPD)], acc.at[pl.ds(s * RPD, RPD)])
    pltpu.sync_copy(dst_hbm.at[w], idx_v)
    for k in range(CH // 16):
        ones_v[pl.ds(k * 16, 16)] = jnp.ones((16,), jnp.float32)
    plsc.subcore_barrier()

    def body(j, carry):
        pltpu.sync_copy(ones_v, acc.at[idx_v.at[j]], add=True)
        return carry

    lax.fori_loop(0, CHUNKS, body, 0)
    plsc.subcore_barrier()
    pltpu.sync_copy(acc.at[pl.ds(s * RPD, RPD)], out_hbm.at[c, pl.ds(s * RPD, RPD)])


def _sc_degree(dst3, z1d):
    kern = pl.kernel(
        _deg_body,
        out_type=jax.ShapeDtypeStruct((NC, NPD), jnp.float32),
        mesh=_mesh(),
        scratch_types=[
            pltpu.VMEM((CHUNKS, CH), jnp.int32),
            pltpu.VMEM((CH,), jnp.float32),
            pltpu.VMEM_SHARED((NPD,), jnp.float32),
        ],
    )
    return kern(dst3, z1d)


def _spmm_body(xh_hbm, src_hbm, dst_hbm, z_hbm, out_hbm,
               src_v, dst_v, gb0, gb1, sem0, sem1, acc):
    c = lax.axis_index("c")
    s = lax.axis_index("s")
    w = c * NS + s
    pltpu.sync_copy(z_hbm.at[pl.ds(s * RPS, RPS)], acc.at[pl.ds(s * RPS, RPS)])
    pltpu.sync_copy(src_hbm.at[w], src_v)
    pltpu.sync_copy(dst_hbm.at[w], dst_v)
    plsc.subcore_barrier()

    pltpu.async_copy(xh_hbm.at[src_v.at[0]], gb0, sem0)

    def body(j, carry):
        @pl.when(j % 2 == 0)
        def _():
            pltpu.make_async_copy(xh_hbm.at[src_v.at[j]], gb0, sem0).wait()

            @pl.when(j + 1 < CHUNKS)
            def _():
                pltpu.async_copy(xh_hbm.at[src_v.at[j + 1]], gb1, sem1)

            pltpu.sync_copy(gb0, acc.at[dst_v.at[j]], add=True)

        @pl.when(j % 2 == 1)
        def _():
            pltpu.make_async_copy(xh_hbm.at[src_v.at[j]], gb1, sem1).wait()

            @pl.when(j + 1 < CHUNKS)
            def _():
                pltpu.async_copy(xh_hbm.at[src_v.at[j + 1]], gb0, sem0)

            pltpu.sync_copy(gb1, acc.at[dst_v.at[j]], add=True)

        return carry

    lax.fori_loop(0, CHUNKS, body, 0)
    plsc.subcore_barrier()
    pltpu.sync_copy(acc.at[pl.ds(s * RPS, RPS)], out_hbm.at[c, pl.ds(s * RPS, RPS)])


def _sc_spmm(xh, src3, dst3, z2d):
    kern = pl.kernel(
        _spmm_body,
        out_type=jax.ShapeDtypeStruct((NC, NP, D), jnp.float32),
        mesh=_mesh(),
        scratch_types=[
            pltpu.VMEM((CHUNKS, CH), jnp.int32),
            pltpu.VMEM((CHUNKS, CH), jnp.int32),
            pltpu.VMEM((CH, D), jnp.float32),
            pltpu.VMEM((CH, D), jnp.float32),
            pltpu.SemaphoreType.DMA,
            pltpu.SemaphoreType.DMA,
            pltpu.VMEM_SHARED((NP, D), jnp.float32),
        ],
    )
    return kern(xh, src3, dst3, z2d)


_BLK = 1000


def _t0_body(degp_ref, x_ref, dinv_ref, xh_ref):
    deg = degp_ref[0, :] + degp_ref[1, :] + 1.0
    dinv = lax.rsqrt(deg)[:, None]
    dinv_ref[...] = dinv
    xh_ref[...] = x_ref[...] * dinv


def _tc_t0(deg_parts, x):
    return pl.pallas_call(
        _t0_body,
        grid=(N // _BLK,),
        in_specs=[
            pl.BlockSpec((NC, _BLK), lambda i: (0, i)),
            pl.BlockSpec((_BLK, D), lambda i: (i, 0)),
        ],
        out_specs=[
            pl.BlockSpec((_BLK, 1), lambda i: (i, 0)),
            pl.BlockSpec((_BLK, D), lambda i: (i, 0)),
        ],
        out_shape=[
            jax.ShapeDtypeStruct((N, 1), jnp.float32),
            jax.ShapeDtypeStruct((N, D), jnp.float32),
        ],
    )(deg_parts, x)


def _layer_body(sp_ref, xh_ref, dinv_ref, w_ref, b_ref, p_ref, xhn_ref):
    dinv = dinv_ref[...]
    p = dinv * (sp_ref[0] + sp_ref[1] + xh_ref[...])
    p_ref[...] = p
    h = jnp.maximum(jnp.dot(p, w_ref[...],
                            preferred_element_type=jnp.float32) + b_ref[...], 0.0)
    xhn_ref[...] = dinv * h


def _tc_layer(s_parts, xh, dinv, W, b):
    return pl.pallas_call(
        _layer_body,
        grid=(N // _BLK,),
        in_specs=[
            pl.BlockSpec((NC, _BLK, D), lambda i: (0, i, 0)),
            pl.BlockSpec((_BLK, D), lambda i: (i, 0)),
            pl.BlockSpec((_BLK, 1), lambda i: (i, 0)),
            pl.BlockSpec((D, D), lambda i: (0, 0)),
            pl.BlockSpec((D,), lambda i: (0,)),
        ],
        out_specs=[
            pl.BlockSpec((_BLK, D), lambda i: (i, 0)),
            pl.BlockSpec((_BLK, D), lambda i: (i, 0)),
        ],
        out_shape=[
            jax.ShapeDtypeStruct((N, D), jnp.float32),
            jax.ShapeDtypeStruct((N, D), jnp.float32),
        ],
    )(s_parts, xh, dinv, W, b)


def _final_body(sp_ref, xh_ref, dinv_ref, p1_ref, p2_ref, wm_ref, bm_ref, out_ref):
    p3 = dinv_ref[...] * (sp_ref[0] + sp_ref[1] + xh_ref[...])
    acc = jnp.dot(p1_ref[...], wm_ref[0:D], preferred_element_type=jnp.float32)
    acc += jnp.dot(p2_ref[...], wm_ref[D:2 * D], preferred_element_type=jnp.float32)
    acc += jnp.dot(p3, wm_ref[2 * D:3 * D], preferred_element_type=jnp.float32)
    out_ref[...] = acc + bm_ref[...]


def _tc_final(s_parts, xh, dinv, p1, p2, Wm, bm):
    return pl.pallas_call(
        _final_body,
        grid=(N // _BLK,),
        in_specs=[
            pl.BlockSpec((NC, _BLK, D), lambda i: (0, i, 0)),
            pl.BlockSpec((_BLK, D), lambda i: (i, 0)),
            pl.BlockSpec((_BLK, 1), lambda i: (i, 0)),
            pl.BlockSpec((_BLK, D), lambda i: (i, 0)),
            pl.BlockSpec((_BLK, D), lambda i: (i, 0)),
            pl.BlockSpec((3 * D, D), lambda i: (0, 0)),
            pl.BlockSpec((D,), lambda i: (0,)),
        ],
        out_specs=pl.BlockSpec((_BLK, D), lambda i: (i, 0)),
        out_shape=jax.ShapeDtypeStruct((N, D), jnp.float32),
    )(s_parts, xh, dinv, p1, p2, Wm, bm)


@jax.jit
def kernel(x, edge_index, W1, b1, W2, b2, W3, b3, Wm, bm):
    src = edge_index[0]
    dst = edge_index[1]
    pad = E_PAD - E
    pad_ids = jnp.arange(pad, dtype=jnp.int32)
    src_pad = jnp.concatenate([src, pad_ids % N])
    dst_pad = jnp.concatenate([dst, N + (pad_ids % 16)])
    src3 = src_pad.reshape(NW, CHUNKS, CH)
    dst3 = dst_pad.reshape(NW, CHUNKS, CH)
    z1d = jnp.zeros((NPD,), jnp.float32)
    z2d = jnp.zeros((NP, D), jnp.float32)

    deg_parts = _sc_degree(dst3, z1d)
    dinv, xh0 = _tc_t0(deg_parts, x)

    s0 = _sc_spmm(xh0, src3, dst3, z2d)[:, :N]
    _, xh1 = _tc_layer(s0, xh0, dinv, W1, b1)
    s1 = _sc_spmm(xh1, src3, dst3, z2d)[:, :N]
    p1, xh2 = _tc_layer(s1, xh1, dinv, W2, b2)
    s2 = _sc_spmm(xh2, src3, dst3, z2d)[:, :N]
    p2, xh3 = _tc_layer(s2, xh2, dinv, W3, b3)
    s3 = _sc_spmm(xh3, src3, dst3, z2d)[:, :N]
    return _tc_final(s3, xh3, dinv, p1, p2, Wm, bm)


# speedup vs baseline: 21.1557x; 21.1557x over previous
import functools

import jax
import jax.numpy as jnp
from jax import lax
from jax.experimental import pallas as pl
from jax.experimental.pallas import tpu as pltpu
from jax.experimental.pallas import tpu_sc as plsc

N = 10000
D = 128
NC = 2
NS = 16
NW = NC * NS
CH = 128
E = 320000
CHUNKS = -(-E // (NW * CH))
E_PAD = NW * CHUNKS * CH
NP = 10112
RPS = NP // NS
NPD = 10240
RPD = NPD // NS

_mesh = lambda: plsc.VectorSubcoreMesh(core_axis_name="c", subcore_axis_name="s")


def _deg_body(dst_hbm, z_hbm, out_hbm, idx_v, ones_v, acc):
    c = lax.axis_index("c")
    s = lax.axis_index("s")
    w = c * NS + s
    pltpu.sync_copy(z_hbm.at[pl.ds(s * RPD, RPD)], acc.at[pl.ds(s * RPD, RPD)])
    pltpu.sync_copy(dst_hbm.at[w], idx_v)
    for k in range(CH // 16):
        ones_v[pl.ds(k * 16, 16)] = jnp.ones((16,), jnp.float32)
    plsc.subcore_barrier()

    def body(j, carry):
        pltpu.sync_copy(ones_v, acc.at[idx_v.at[j]], add=True)
        return carry

    lax.fori_loop(0, CHUNKS, body, 0)
    plsc.subcore_barrier()
    pltpu.sync_copy(acc.at[pl.ds(s * RPD, RPD)], out_hbm.at[c, pl.ds(s * RPD, RPD)])


def _sc_degree(dst3, z1d):
    kern = pl.kernel(
        _deg_body,
        out_type=jax.ShapeDtypeStruct((NC, NPD), jnp.float32),
        mesh=_mesh(),
        scratch_types=[
            pltpu.VMEM((CHUNKS, CH), jnp.int32),
            pltpu.VMEM((CH,), jnp.float32),
            pltpu.VMEM_SHARED((NPD,), jnp.float32),
        ],
    )
    return kern(dst3, z1d)


def _spmm_body(xh_hbm, src_hbm, dst_hbm, z_hbm, out_hbm,
               src_v, dst_v, gb0, gb1, sem0, sem1, acc):
    c = lax.axis_index("c")
    s = lax.axis_index("s")
    w = c * NS + s
    pltpu.sync_copy(z_hbm.at[pl.ds(s * RPS, RPS)], acc.at[pl.ds(s * RPS, RPS)])
    plsc.subcore_barrier()

    pltpu.sync_copy(src_hbm.at[w, 0], src_v.at[0])
    pltpu.async_copy(xh_hbm.at[src_v.at[0]], gb0, sem0)

    def body(j, carry):
        @pl.when(j % 2 == 0)
        def _():
            pltpu.make_async_copy(xh_hbm.at[src_v.at[0]], gb0, sem0).wait()

            @pl.when(j + 1 < CHUNKS)
            def _():
                pltpu.sync_copy(src_hbm.at[w, j + 1], src_v.at[1])
                pltpu.async_copy(xh_hbm.at[src_v.at[1]], gb1, sem1)

            pltpu.sync_copy(dst_hbm.at[w, j], dst_v.at[0])
            pltpu.sync_copy(gb0, acc.at[dst_v.at[0]], add=True)

        @pl.when(j % 2 == 1)
        def _():
            pltpu.make_async_copy(xh_hbm.at[src_v.at[1]], gb1, sem1).wait()

            @pl.when(j + 1 < CHUNKS)
            def _():
                pltpu.sync_copy(src_hbm.at[w, j + 1], src_v.at[0])
                pltpu.async_copy(xh_hbm.at[src_v.at[0]], gb0, sem0)

            pltpu.sync_copy(dst_hbm.at[w, j], dst_v.at[0])
            pltpu.sync_copy(gb1, acc.at[dst_v.at[0]], add=True)

        return carry

    lax.fori_loop(0, CHUNKS, body, 0)
    plsc.subcore_barrier()
    pltpu.sync_copy(acc.at[pl.ds(s * RPS, RPS)], out_hbm.at[c, pl.ds(s * RPS, RPS)])


def _sc_spmm(xh, src3, dst3, z2d):
    kern = pl.kernel(
        _spmm_body,
        out_type=jax.ShapeDtypeStruct((NC, NP, D), jnp.float32),
        mesh=_mesh(),
        scratch_types=[
            pltpu.VMEM((2, CH), jnp.int32),
            pltpu.VMEM((1, CH), jnp.int32),
            pltpu.VMEM((CH, D), jnp.float32),
            pltpu.VMEM((CH, D), jnp.float32),
            pltpu.SemaphoreType.DMA,
            pltpu.SemaphoreType.DMA,
            pltpu.VMEM_SHARED((NP, D), jnp.float32),
        ],
    )
    return kern(xh, src3, dst3, z2d)


_BLK = 1000


def _t0_body(degp_ref, x_ref, dinv_ref, xh_ref):
    deg = degp_ref[0] + degp_ref[1] + 1.0
    dinv = lax.rsqrt(deg)
    dinv_ref[...] = dinv
    xh_ref[...] = x_ref[...] * dinv


def _tc_t0(deg_parts, x):
    return pl.pallas_call(
        _t0_body,
        grid=(N // _BLK,),
        in_specs=[
            pl.BlockSpec((NC, _BLK, 1), lambda i: (0, i, 0)),
            pl.BlockSpec((_BLK, D), lambda i: (i, 0)),
        ],
        out_specs=[
            pl.BlockSpec((_BLK, 1), lambda i: (i, 0)),
            pl.BlockSpec((_BLK, D), lambda i: (i, 0)),
        ],
        out_shape=[
            jax.ShapeDtypeStruct((N, 1), jnp.float32),
            jax.ShapeDtypeStruct((N, D), jnp.float32),
        ],
    )(deg_parts, x)


def _layer_body(sp_ref, xh_ref, dinv_ref, w_ref, b_ref, p_ref, xhn_ref):
    dinv = dinv_ref[...]
    p = dinv * (sp_ref[0] + sp_ref[1] + xh_ref[...])
    p_ref[...] = p
    h = jnp.maximum(jnp.dot(p, w_ref[...],
                            preferred_element_type=jnp.float32) + b_ref[...], 0.0)
    xhn_ref[...] = dinv * h


def _tc_layer(s_parts, xh, dinv, W, b):
    return pl.pallas_call(
        _layer_body,
        grid=(N // _BLK,),
        in_specs=[
            pl.BlockSpec((NC, _BLK, D), lambda i: (0, i, 0)),
            pl.BlockSpec((_BLK, D), lambda i: (i, 0)),
            pl.BlockSpec((_BLK, 1), lambda i: (i, 0)),
            pl.BlockSpec((D, D), lambda i: (0, 0)),
            pl.BlockSpec((D,), lambda i: (0,)),
        ],
        out_specs=[
            pl.BlockSpec((_BLK, D), lambda i: (i, 0)),
            pl.BlockSpec((_BLK, D), lambda i: (i, 0)),
        ],
        out_shape=[
            jax.ShapeDtypeStruct((N, D), jnp.float32),
            jax.ShapeDtypeStruct((N, D), jnp.float32),
        ],
    )(s_parts, xh, dinv, W, b)


def _final_body(sp_ref, xh_ref, dinv_ref, p1_ref, p2_ref, wm_ref, bm_ref, out_ref):
    p3 = dinv_ref[...] * (sp_ref[0] + sp_ref[1] + xh_ref[...])
    acc = jnp.dot(p1_ref[...], wm_ref[0:D], preferred_element_type=jnp.float32)
    acc += jnp.dot(p2_ref[...], wm_ref[D:2 * D], preferred_element_type=jnp.float32)
    acc += jnp.dot(p3, wm_ref[2 * D:3 * D], preferred_element_type=jnp.float32)
    out_ref[...] = acc + bm_ref[...]


def _tc_final(s_parts, xh, dinv, p1, p2, Wm, bm):
    return pl.pallas_call(
        _final_body,
        grid=(N // _BLK,),
        in_specs=[
            pl.BlockSpec((NC, _BLK, D), lambda i: (0, i, 0)),
            pl.BlockSpec((_BLK, D), lambda i: (i, 0)),
            pl.BlockSpec((_BLK, 1), lambda i: (i, 0)),
            pl.BlockSpec((_BLK, D), lambda i: (i, 0)),
            pl.BlockSpec((_BLK, D), lambda i: (i, 0)),
            pl.BlockSpec((3 * D, D), lambda i: (0, 0)),
            pl.BlockSpec((D,), lambda i: (0,)),
        ],
        out_specs=pl.BlockSpec((_BLK, D), lambda i: (i, 0)),
        out_shape=jax.ShapeDtypeStruct((N, D), jnp.float32),
    )(s_parts, xh, dinv, p1, p2, Wm, bm)


@jax.jit
def kernel(x, edge_index, W1, b1, W2, b2, W3, b3, Wm, bm):
    src = edge_index[0]
    dst = edge_index[1]
    pad = E_PAD - E
    pad_ids = jnp.arange(pad, dtype=jnp.int32)
    src_pad = jnp.concatenate([src, pad_ids % N])
    dst_pad = jnp.concatenate([dst, N + (pad_ids % 16)])
    src3 = src_pad.reshape(NW, CHUNKS, CH)
    dst3 = dst_pad.reshape(NW, CHUNKS, CH)
    z1d = jnp.zeros((NPD,), jnp.float32)
    z2d = jnp.zeros((NP, D), jnp.float32)

    deg_parts = _sc_degree(dst3, z1d).reshape(NC, NPD, 1)
    dinv, xh0 = _tc_t0(deg_parts, x)

    s0 = _sc_spmm(xh0, src3, dst3, z2d)
    _, xh1 = _tc_layer(s0, xh0, dinv, W1, b1)
    s1 = _sc_spmm(xh1, src3, dst3, z2d)
    p1, xh2 = _tc_layer(s1, xh1, dinv, W2, b2)
    s2 = _sc_spmm(xh2, src3, dst3, z2d)
    p2, xh3 = _tc_layer(s2, xh2, dinv, W3, b3)
    s3 = _sc_spmm(xh3, src3, dst3, z2d)
    return _tc_final(s3, xh3, dinv, p1, p2, Wm, bm)


# speedup vs baseline: 25.7649x; 1.2179x over previous
import functools

import jax
import jax.numpy as jnp
from jax import lax
from jax.experimental import pallas as pl
from jax.experimental.pallas import tpu as pltpu
from jax.experimental.pallas import tpu_sc as plsc

N = 10000
D = 128
NC = 2
NS = 16
NW = NC * NS
CH = 128
E = 320000
CHUNKS = -(-E // (NW * CH))
E_PAD = NW * CHUNKS * CH
NP = 10112
RPS = NP // NS
NPD = 10240
RPD = NPD // NS

_mesh = lambda: plsc.VectorSubcoreMesh(core_axis_name="c", subcore_axis_name="s")


def _deg_body(dst_hbm, z_hbm, out_hbm, idx_v, ones_v, acc):
    c = lax.axis_index("c")
    s = lax.axis_index("s")
    w = c * NS + s
    pltpu.sync_copy(z_hbm.at[pl.ds(s * RPD, RPD)], acc.at[pl.ds(s * RPD, RPD)])
    pltpu.sync_copy(dst_hbm.at[w], idx_v)
    for k in range(CH // 16):
        ones_v[pl.ds(k * 16, 16)] = jnp.ones((16,), jnp.float32)
    plsc.subcore_barrier()

    def body(j, carry):
        pltpu.sync_copy(ones_v, acc.at[idx_v.at[j]], add=True)
        return carry

    lax.fori_loop(0, CHUNKS, body, 0)
    plsc.subcore_barrier()
    pltpu.sync_copy(acc.at[pl.ds(s * RPD, RPD)], out_hbm.at[c, pl.ds(s * RPD, RPD)])


def _sc_degree(dst3, z1d):
    kern = pl.kernel(
        _deg_body,
        out_type=jax.ShapeDtypeStruct((NC, NPD), jnp.float32),
        mesh=_mesh(),
        scratch_types=[
            pltpu.VMEM((CHUNKS, CH), jnp.int32),
            pltpu.VMEM((CH,), jnp.float32),
            pltpu.VMEM_SHARED((NPD,), jnp.float32),
        ],
    )
    return kern(dst3, z1d)


def _spmm_body(xh_hbm, src_hbm, dst_hbm, z_hbm, out_hbm,
               src_v, dst_v, gb0, gb1,
               sem_g0, sem_g1, sem_s0, sem_s1, sem_d0, sem_d1, acc):
    c = lax.axis_index("c")
    s = lax.axis_index("s")
    w = c * NS + s
    pltpu.sync_copy(z_hbm.at[pl.ds(s * RPS, RPS)], acc.at[pl.ds(s * RPS, RPS)])
    plsc.subcore_barrier()

    pltpu.sync_copy(src_hbm.at[w, 0], src_v.at[0])
    pltpu.async_copy(xh_hbm.at[src_v.at[0]], gb0, sem_g0)
    pltpu.async_copy(src_hbm.at[w, 1], src_v.at[1], sem_s1)
    pltpu.async_copy(dst_hbm.at[w, 0], dst_v.at[0], sem_d0)
    pltpu.async_copy(dst_hbm.at[w, 1], dst_v.at[1], sem_d1)

    def step(j, gb_c, sem_gc, gb_n, sem_gn, src_c, src_n, sem_sn, sem_sc,
             dst_c, sem_dc):
        pltpu.make_async_copy(xh_hbm.at[src_c], gb_c, sem_gc).wait()

        @pl.when(j + 1 < CHUNKS)
        def _():
            pltpu.make_async_copy(src_hbm.at[w, 0], src_n, sem_sn).wait()
            pltpu.async_copy(xh_hbm.at[src_n], gb_n, sem_gn)

        @pl.when(j + 2 < CHUNKS)
        def _():
            pltpu.async_copy(src_hbm.at[w, j + 2], src_c, sem_sc)

        pltpu.make_async_copy(dst_hbm.at[w, 0], dst_c, sem_dc).wait()
        pltpu.sync_copy(gb_c, acc.at[dst_c], add=True)

        @pl.when(j + 2 < CHUNKS)
        def _():
            pltpu.async_copy(dst_hbm.at[w, j + 2], dst_c, sem_dc)

    def body(j, carry):
        @pl.when(j % 2 == 0)
        def _():
            step(j, gb0, sem_g0, gb1, sem_g1, src_v.at[0], src_v.at[1],
                 sem_s1, sem_s0, dst_v.at[0], sem_d0)

        @pl.when(j % 2 == 1)
        def _():
            step(j, gb1, sem_g1, gb0, sem_g0, src_v.at[1], src_v.at[0],
                 sem_s0, sem_s1, dst_v.at[1], sem_d1)

        return carry

    lax.fori_loop(0, CHUNKS, body, 0)
    plsc.subcore_barrier()
    pltpu.sync_copy(acc.at[pl.ds(s * RPS, RPS)], out_hbm.at[c, pl.ds(s * RPS, RPS)])


def _sc_spmm(xh, src3, dst3, z2d):
    kern = pl.kernel(
        _spmm_body,
        out_type=jax.ShapeDtypeStruct((NC, NP, D), jnp.float32),
        mesh=_mesh(),
        scratch_types=[
            pltpu.VMEM((2, CH), jnp.int32),
            pltpu.VMEM((2, CH), jnp.int32),
            pltpu.VMEM((CH, D), jnp.float32),
            pltpu.VMEM((CH, D), jnp.float32),
            pltpu.SemaphoreType.DMA,
            pltpu.SemaphoreType.DMA,
            pltpu.SemaphoreType.DMA,
            pltpu.SemaphoreType.DMA,
            pltpu.SemaphoreType.DMA,
            pltpu.SemaphoreType.DMA,
            pltpu.VMEM_SHARED((NP, D), jnp.float32),
        ],
    )
    return kern(xh, src3, dst3, z2d)


_BLK = 1000


def _t0_body(degp_ref, x_ref, dinv_ref, xh_ref):
    deg = degp_ref[0] + degp_ref[1] + 1.0
    dinv = lax.rsqrt(deg)
    dinv_ref[...] = dinv
    xh_ref[...] = x_ref[...] * dinv


def _tc_t0(deg_parts, x):
    return pl.pallas_call(
        _t0_body,
        grid=(N // _BLK,),
        in_specs=[
            pl.BlockSpec((NC, _BLK, 1), lambda i: (0, i, 0)),
            pl.BlockSpec((_BLK, D), lambda i: (i, 0)),
        ],
        out_specs=[
            pl.BlockSpec((_BLK, 1), lambda i: (i, 0)),
            pl.BlockSpec((_BLK, D), lambda i: (i, 0)),
        ],
        out_shape=[
            jax.ShapeDtypeStruct((N, 1), jnp.float32),
            jax.ShapeDtypeStruct((N, D), jnp.float32),
        ],
    )(deg_parts, x)


def _layer_body(sp_ref, xh_ref, dinv_ref, w_ref, b_ref, p_ref, xhn_ref):
    dinv = dinv_ref[...]
    p = dinv * (sp_ref[0] + sp_ref[1] + xh_ref[...])
    p_ref[...] = p
    h = jnp.maximum(jnp.dot(p, w_ref[...],
                            preferred_element_type=jnp.float32) + b_ref[...], 0.0)
    xhn_ref[...] = dinv * h


def _tc_layer(s_parts, xh, dinv, W, b):
    return pl.pallas_call(
        _layer_body,
        grid=(N // _BLK,),
        in_specs=[
            pl.BlockSpec((NC, _BLK, D), lambda i: (0, i, 0)),
            pl.BlockSpec((_BLK, D), lambda i: (i, 0)),
            pl.BlockSpec((_BLK, 1), lambda i: (i, 0)),
            pl.BlockSpec((D, D), lambda i: (0, 0)),
            pl.BlockSpec((D,), lambda i: (0,)),
        ],
        out_specs=[
            pl.BlockSpec((_BLK, D), lambda i: (i, 0)),
            pl.BlockSpec((_BLK, D), lambda i: (i, 0)),
        ],
        out_shape=[
            jax.ShapeDtypeStruct((N, D), jnp.float32),
            jax.ShapeDtypeStruct((N, D), jnp.float32),
        ],
    )(s_parts, xh, dinv, W, b)


def _final_body(sp_ref, xh_ref, dinv_ref, p1_ref, p2_ref, wm_ref, bm_ref, out_ref):
    p3 = dinv_ref[...] * (sp_ref[0] + sp_ref[1] + xh_ref[...])
    acc = jnp.dot(p1_ref[...], wm_ref[0:D], preferred_element_type=jnp.float32)
    acc += jnp.dot(p2_ref[...], wm_ref[D:2 * D], preferred_element_type=jnp.float32)
    acc += jnp.dot(p3, wm_ref[2 * D:3 * D], preferred_element_type=jnp.float32)
    out_ref[...] = acc + bm_ref[...]


def _tc_final(s_parts, xh, dinv, p1, p2, Wm, bm):
    return pl.pallas_call(
        _final_body,
        grid=(N // _BLK,),
        in_specs=[
            pl.BlockSpec((NC, _BLK, D), lambda i: (0, i, 0)),
            pl.BlockSpec((_BLK, D), lambda i: (i, 0)),
            pl.BlockSpec((_BLK, 1), lambda i: (i, 0)),
            pl.BlockSpec((_BLK, D), lambda i: (i, 0)),
            pl.BlockSpec((_BLK, D), lambda i: (i, 0)),
            pl.BlockSpec((3 * D, D), lambda i: (0, 0)),
            pl.BlockSpec((D,), lambda i: (0,)),
        ],
        out_specs=pl.BlockSpec((_BLK, D), lambda i: (i, 0)),
        out_shape=jax.ShapeDtypeStruct((N, D), jnp.float32),
    )(s_parts, xh, dinv, p1, p2, Wm, bm)


@jax.jit
def kernel(x, edge_index, W1, b1, W2, b2, W3, b3, Wm, bm):
    src = edge_index[0]
    dst = edge_index[1]
    pad = E_PAD - E
    pad_ids = jnp.arange(pad, dtype=jnp.int32)
    src_pad = jnp.concatenate([src, pad_ids % N])
    dst_pad = jnp.concatenate([dst, N + (pad_ids % 16)])
    src3 = src_pad.reshape(NW, CHUNKS, CH)
    dst3 = dst_pad.reshape(NW, CHUNKS, CH)
    z1d = jnp.zeros((NPD,), jnp.float32)
    z2d = jnp.zeros((NP, D), jnp.float32)

    deg_parts = _sc_degree(dst3, z1d).reshape(NC, NPD, 1)
    dinv, xh0 = _tc_t0(deg_parts, x)

    s0 = _sc_spmm(xh0, src3, dst3, z2d)
    _, xh1 = _tc_layer(s0, xh0, dinv, W1, b1)
    s1 = _sc_spmm(xh1, src3, dst3, z2d)
    p1, xh2 = _tc_layer(s1, xh1, dinv, W2, b2)
    s2 = _sc_spmm(xh2, src3, dst3, z2d)
    p2, xh3 = _tc_layer(s2, xh2, dinv, W3, b3)
    s3 = _sc_spmm(xh3, src3, dst3, z2d)
    return _tc_final(s3, xh3, dinv, p1, p2, Wm, bm)
